# Initial kernel scaffold; baseline (speedup 1.0000x reference)
import jax
import jax.numpy as jnp
from jax.experimental import pallas as pl


def kernel(state, internal, edge_index, W_enc, b_enc):
    raise NotImplementedError("write your pallas kernel here")


# speedup vs baseline: 5.3022x; 5.3022x over previous
import functools

import jax
import jax.numpy as jnp
from jax import lax
from jax.experimental import pallas as pl
from jax.experimental.pallas import tpu as pltpu
from jax.experimental.pallas import tpu_sc as plsc


def _encode(x, w, b2d, block_rows=1000):
  m, d = x.shape
  h = w.shape[1]

  def body(x_ref, w_ref, b_ref, o_ref):
    acc = jnp.dot(x_ref[...], w_ref[...], preferred_element_type=jnp.float32)
    o_ref[...] = jnp.maximum(acc + b_ref[...], 0.0)

  return pl.pallas_call(
      body,
      grid=(m // block_rows,),
      in_specs=[
          pl.BlockSpec((block_rows, d), lambda i: (i, 0)),
          pl.BlockSpec((d, h), lambda i: (0, 0)),
          pl.BlockSpec((1, h), lambda i: (0, 0)),
      ],
      out_specs=pl.BlockSpec((block_rows, h), lambda i: (i, 0)),
      out_shape=jax.ShapeDtypeStruct((m, h), jnp.float32),
  )(x, w, b2d)


def _combine(partial, n, block_rows=1000):
  _, _, d = partial.shape

  def body(p_ref, o_ref):
    o_ref[...] = p_ref[0] + p_ref[1]

  return pl.pallas_call(
      body,
      grid=(n // block_rows,),
      in_specs=[pl.BlockSpec((2, block_rows, d), lambda i: (0, i, 0))],
      out_specs=pl.BlockSpec((block_rows, d), lambda i: (i, 0)),
      out_shape=jax.ShapeDtypeStruct((n, d), jnp.float32),
  )(partial)


def _edge_aggregate(embed, src, dst, zeros_tile, n_pad):
  n, d = embed.shape
  e = src.shape[0]
  num_cores = 2
  num_subcores = 16
  nw = num_cores * num_subcores
  edges_per_tile = e // nw
  chunk = 80
  nchunks = edges_per_tile // chunk
  rows_per_tile = n_pad // num_subcores

  mesh = plsc.VectorSubcoreMesh(core_axis_name="c", subcore_axis_name="s")

  @functools.partial(
      pl.kernel,
      mesh=mesh,
      out_type=jax.ShapeDtypeStruct((num_cores, n_pad, d), jnp.float32),
      scratch_types=[
          pltpu.VMEM_SHARED((n_pad, d), jnp.float32),
          pltpu.VMEM((chunk,), jnp.int32),
          pltpu.VMEM((chunk,), jnp.int32),
          pltpu.VMEM((chunk, d), jnp.float32),
          pltpu.SemaphoreType.DMA,
      ],
  )
  def agg(embed_hbm, src_hbm, dst_hbm, zero_hbm, out_hbm,
          acc, src_idx, dst_idx, rows, sem):
    c = lax.axis_index("c")
    s = lax.axis_index("s")
    wid = c * num_subcores + s

    pltpu.sync_copy(zero_hbm, acc.at[pl.ds(s * rows_per_tile, rows_per_tile)])
    plsc.subcore_barrier()

    base = wid * edges_per_tile

    @pl.loop(0, nchunks)
    def _(g):
      off = base + g * chunk
      pltpu.sync_copy(src_hbm.at[pl.ds(off, chunk)], src_idx)
      pltpu.sync_copy(dst_hbm.at[pl.ds(off, chunk)], dst_idx)
      pltpu.async_copy(embed_hbm.at[src_idx], rows, sem).wait()
      pltpu.sync_copy(rows, acc.at[dst_idx], add=True)

    plsc.subcore_barrier()
    sl = pl.ds(s * rows_per_tile, rows_per_tile)
    pltpu.sync_copy(acc.at[sl], out_hbm.at[c, sl])

  return agg(embed, src, dst, zeros_tile)


def kernel(state, internal, edge_index, W_enc, b_enc):
  n, d = state.shape
  n_pad = ((n + 127) // 128) * 128
  b2d = b_enc.reshape(1, -1)
  src = edge_index[0]
  dst = edge_index[1]
  zeros_tile = jnp.zeros((n_pad // 16, d), jnp.float32)

  state_embed = _encode(state, W_enc, b2d)
  partial = _edge_aggregate(state_embed, src, dst, zeros_tile, n_pad)
  internal_embed = _encode(internal, W_enc, b2d)
  neigh_sum = _combine(partial, n)
  return (state_embed, internal_embed, neigh_sum)
